# baseline (speedup 1.0000x reference)
import functools

import jax
import jax.numpy as jnp
from jax import lax
from jax.experimental import pallas as pl
from jax.experimental.pallas import tpu as pltpu
from jax.experimental.pallas import tpu_sc as plsc


@functools.lru_cache(maxsize=None)
def _make_gather(V, D, B):
    info = plsc.get_sparse_core_info()
    nc, ns = info.num_cores, info.num_subcores
    nw = nc * ns
    assert B % (8 * nw) == 0, (V, D, B)
    b_per_w = B // nw
    nch = 4
    nbuf = 2
    assert b_per_w % nch == 0
    ch = b_per_w // nch
    mesh = plsc.VectorSubcoreMesh(core_axis_name="c", subcore_axis_name="s")

    @functools.partial(
        pl.kernel,
        mesh=mesh,
        out_type=jax.ShapeDtypeStruct((B, D), jnp.float32),
        scratch_types=[
            pltpu.VMEM((b_per_w,), jnp.int32),
            pltpu.VMEM((b_per_w, D), jnp.float32),
            pltpu.VMEM_SHARED((ns, nbuf, ch, D), jnp.float32),
        ]
        + [pltpu.SemaphoreType.DMA] * (nch + 2 * nbuf),
    )
    def gather_kernel(table_hbm, idx_hbm, out_hbm, idx_v, rows_v, shared, *sems):
        gsems = sems[:nch]
        csems = sems[nch : nch + nbuf]
        dsems = sems[nch + nbuf :]
        cid = lax.axis_index("c")
        sid = lax.axis_index("s")
        wid = sid * nc + cid
        base = wid * b_per_w
        pltpu.sync_copy(idx_hbm.at[pl.ds(base, b_per_w)], idx_v)
        gathers = [
            pltpu.async_copy(
                table_hbm.at[idx_v.at[pl.ds(k * ch, ch)]],
                rows_v.at[pl.ds(k * ch, ch)],
                gsems[k],
            )
            for k in range(nch)
        ]
        dmas = [None] * nch
        for k in range(nch):
            gathers[k].wait()
            if k >= nbuf:
                dmas[k - nbuf].wait()
            b = k % nbuf
            pltpu.async_copy(
                rows_v.at[pl.ds(k * ch, ch)], shared.at[sid].at[b], csems[b]
            ).wait()
            dmas[k] = pltpu.async_copy(
                shared.at[sid].at[b],
                out_hbm.at[pl.ds(base + k * ch, ch)],
                dsems[b],
            )
        for k in range(nch - nbuf, nch):
            dmas[k].wait()

    return gather_kernel


def kernel(x, n):
    V, D = x.shape
    (B,) = n.shape
    return _make_gather(V, D, B)(x, n.astype(jnp.int32))


# speedup vs baseline: 1.0590x; 1.0590x over previous
import functools

import jax
import jax.numpy as jnp
from jax import lax
from jax.experimental import pallas as pl
from jax.experimental.pallas import tpu as pltpu
from jax.experimental.pallas import tpu_sc as plsc


@functools.lru_cache(maxsize=None)
def _make_gather(V, D, B):
    info = plsc.get_sparse_core_info()
    nc, ns = info.num_cores, info.num_subcores
    nw = nc * ns
    assert B % (8 * nw) == 0, (V, D, B)
    b_per_w = B // nw
    nch = 4
    nsp = 2
    assert b_per_w % nch == 0
    ch = b_per_w // nch
    mesh = plsc.VectorSubcoreMesh(core_axis_name="c", subcore_axis_name="s")

    @functools.partial(
        pl.kernel,
        mesh=mesh,
        out_type=jax.ShapeDtypeStruct((B, D), jnp.float32),
        scratch_types=[
            pltpu.VMEM((b_per_w,), jnp.int32),
            pltpu.VMEM((b_per_w, D), jnp.float32),
            pltpu.VMEM_SHARED((ns, nsp, ch, D), jnp.float32),
        ]
        + [pltpu.SemaphoreType.DMA] * (nch + 2 * nsp + 1),
    )
    def gather_kernel(table_hbm, idx_hbm, out_hbm, idx_v, rows_v, shared, *sems):
        gsems = sems[:nch]
        csems = sems[nch : nch + nsp]
        dsems = sems[nch + nsp : nch + 2 * nsp]
        wsem = sems[-1]
        cid = lax.axis_index("c")
        sid = lax.axis_index("s")
        wid = sid * nc + cid
        base = wid * b_per_w
        pltpu.sync_copy(idx_hbm.at[pl.ds(base, b_per_w)], idx_v)
        gathers = [
            pltpu.async_copy(
                table_hbm.at[idx_v.at[pl.ds(k * ch, ch)]],
                rows_v.at[pl.ds(k * ch, ch)],
                gsems[k],
            )
            for k in range(nch)
        ]
        dmas = []
        for k in range(nsp):
            gathers[k].wait()
            pltpu.async_copy(
                rows_v.at[pl.ds(k * ch, ch)], shared.at[sid].at[k], csems[k]
            ).wait()
            dmas.append(
                pltpu.async_copy(
                    shared.at[sid].at[k],
                    out_hbm.at[pl.ds(base + k * ch, ch)],
                    dsems[k],
                )
            )
        for k in range(nsp, nch):
            gathers[k].wait()
        direct = pltpu.async_copy(
            rows_v.at[pl.ds(nsp * ch, (nch - nsp) * ch)],
            out_hbm.at[pl.ds(base + nsp * ch, (nch - nsp) * ch)],
            wsem,
        )
        for d in dmas:
            d.wait()
        direct.wait()

    return gather_kernel


def kernel(x, n):
    V, D = x.shape
    (B,) = n.shape
    return _make_gather(V, D, B)(x, n.astype(jnp.int32))


# speedup vs baseline: 1.1013x; 1.0399x over previous
import functools

import jax
import jax.numpy as jnp
from jax import lax
from jax.experimental import pallas as pl
from jax.experimental.pallas import tpu as pltpu
from jax.experimental.pallas import tpu_sc as plsc


@functools.lru_cache(maxsize=None)
def _make_gather(V, D, B):
    info = plsc.get_sparse_core_info()
    nc, ns = info.num_cores, info.num_subcores
    nw = nc * ns
    assert B % (8 * nw) == 0, (V, D, B)
    b_per_w = B // nw
    mesh = plsc.VectorSubcoreMesh(core_axis_name="c", subcore_axis_name="s")

    @functools.partial(
        pl.kernel,
        mesh=mesh,
        out_type=jax.ShapeDtypeStruct((B, D), jnp.float32),
        scratch_types=[
            pltpu.VMEM((b_per_w,), jnp.int32),
            pltpu.VMEM((b_per_w, D), jnp.float32),
            pltpu.SemaphoreType.DMA,
        ],
    )
    def gather_kernel(table_hbm, idx_hbm, out_hbm, idx_v, rows_v, sem):
        wid = lax.axis_index("s") * nc + lax.axis_index("c")
        base = wid * b_per_w
        pltpu.sync_copy(idx_hbm.at[pl.ds(base, b_per_w)], idx_v)
        pltpu.async_copy(table_hbm.at[idx_v], rows_v, sem).wait()
        pltpu.sync_copy(rows_v, out_hbm.at[pl.ds(base, b_per_w)])

    return gather_kernel


def kernel(x, n):
    V, D = x.shape
    (B,) = n.shape
    return _make_gather(V, D, B)(x, n.astype(jnp.int32))
